# baseline (speedup 1.0000x reference)
import dataclasses

import jax
import jax.numpy as jnp
from jax import lax
from jax.experimental import pallas as pl
from jax.experimental.pallas import tpu as pltpu
from jax.experimental.pallas import tpu_sc as plsc

_T = 64
_N_PAD = 114688
_CHUNK = 57344
_W_CHUNK = 1792
_TC_B = 8192


def _make_sc_gather(k):
    mesh = plsc.VectorSubcoreMesh(core_axis_name="c", subcore_axis_name="s")
    out_t = jax.ShapeDtypeStruct((1, _CHUNK), jnp.float32)
    cp = pltpu.CompilerParams()
    if "needs_layout_passes" in pltpu.CompilerParams.__dataclass_fields__:
        cp = dataclasses.replace(cp, needs_layout_passes=False)

    @pl.kernel(
        out_type=(out_t, out_t), mesh=mesh, compiler_params=cp,
        scratch_types=[
            pltpu.VMEM((_W_CHUNK,), jnp.int32),
            pltpu.VMEM((1, _T), jnp.float32),
            pltpu.VMEM((1, _T), jnp.float32),
            pltpu.VMEM((_W_CHUNK,), jnp.float32),
            pltpu.VMEM((_W_CHUNK,), jnp.float32),
        ],
    )
    def sc_kernel(idx_hbm, sct_hbm, sht_hbm, scale_hbm, shift_hbm,
                  idx_v, sct_v, sht_v, scale_v, shift_v):
        wid = lax.axis_index("s") * 2 + lax.axis_index("c")
        src = k * _CHUNK + wid * _W_CHUNK
        dst = wid * _W_CHUNK
        pltpu.sync_copy(idx_hbm.at[0, pl.ds(src, _W_CHUNK)], idx_v)
        pltpu.sync_copy(sct_hbm, sct_v)
        pltpu.sync_copy(sht_hbm, sht_v)
        z16 = jnp.zeros((16,), jnp.int32)

        @pl.loop(0, _W_CHUNK, step=16)
        def _(i):
            i16 = idx_v[pl.ds(i, 16)]
            scale_v[pl.ds(i, 16)] = plsc.load_gather(sct_v, [z16, i16])
            shift_v[pl.ds(i, 16)] = plsc.load_gather(sht_v, [z16, i16])

        pltpu.sync_copy(scale_v, scale_hbm.at[0, pl.ds(dst, _W_CHUNK)])
        pltpu.sync_copy(shift_v, shift_hbm.at[0, pl.ds(dst, _W_CHUNK)])

    return sc_kernel


def _tc_apply0(sc_ref, sh_ref, x_ref, o_ref):
    d = x_ref.shape[1]
    ones = jnp.ones((1, d), jnp.float32)
    dn = (((0,), (0,)), ((), ()))
    scale = jax.lax.dot_general(sc_ref[...], ones, dn,
                                preferred_element_type=jnp.float32)
    shift = jax.lax.dot_general(sh_ref[...], ones, dn,
                                preferred_element_type=jnp.float32)
    o_ref[...] = scale * x_ref[...] + shift


def _tc_apply1(prev_ref, sc_ref, sh_ref, x_ref, o_ref):
    del prev_ref
    _tc_apply0(sc_ref, sh_ref, x_ref, o_ref)


def kernel(in_field, species_idx, scales, shifts):
    n, d = in_field.shape
    idx_pad = jnp.pad(species_idx.astype(jnp.int32), (0, _N_PAD - n)).reshape(1, _N_PAD)
    sct = scales.reshape(1, _T)
    sht = shifts.reshape(1, _T)
    sc0, sh0 = _make_sc_gather(0)(idx_pad, sct, sht)
    sc1, sh1 = _make_sc_gather(1)(idx_pad, sct, sht)

    nb0 = _CHUNK // _TC_B
    out0 = pl.pallas_call(
        _tc_apply0,
        grid=(nb0,),
        in_specs=[
            pl.BlockSpec((1, _TC_B), lambda i: (0, i)),
            pl.BlockSpec((1, _TC_B), lambda i: (0, i)),
            pl.BlockSpec((_TC_B, d), lambda i: (i, 0)),
        ],
        out_specs=pl.BlockSpec((_TC_B, d), lambda i: (i, 0)),
        out_shape=jax.ShapeDtypeStruct((n, d), in_field.dtype),
        compiler_params=pltpu.CompilerParams(
            dimension_semantics=("arbitrary",),
        ),
    )(sc0, sh0, in_field)

    nb1 = (n - _CHUNK + _TC_B - 1) // _TC_B
    off = _CHUNK // _TC_B
    return pl.pallas_call(
        _tc_apply1,
        grid=(nb1,),
        in_specs=[
            pl.BlockSpec((8, 128), lambda i: (0, 0)),
            pl.BlockSpec((1, _TC_B), lambda i: (0, i)),
            pl.BlockSpec((1, _TC_B), lambda i: (0, i)),
            pl.BlockSpec((_TC_B, d), lambda i: (i + off, 0)),
        ],
        out_specs=pl.BlockSpec((_TC_B, d), lambda i: (i + off, 0)),
        out_shape=jax.ShapeDtypeStruct((n, d), in_field.dtype),
        input_output_aliases={0: 0},
        compiler_params=pltpu.CompilerParams(
            dimension_semantics=("arbitrary",),
        ),
    )(out0, sc1, sh1, in_field)


# speedup vs baseline: 1.1017x; 1.1017x over previous
import dataclasses

import jax
import jax.numpy as jnp
from jax import lax
from jax.experimental import pallas as pl
from jax.experimental.pallas import tpu as pltpu
from jax.experimental.pallas import tpu_sc as plsc

_T = 64
_N_PAD = 102400
_W_CHUNK = 3200
_TC_B = 20480


def _sc_gather(idx_pad, sct, sht):
    mesh = plsc.VectorSubcoreMesh(core_axis_name="c", subcore_axis_name="s")
    out_t = jax.ShapeDtypeStruct((2, _N_PAD), jnp.float32)
    cp = pltpu.CompilerParams()
    if "needs_layout_passes" in pltpu.CompilerParams.__dataclass_fields__:
        cp = dataclasses.replace(cp, needs_layout_passes=False)

    @pl.kernel(
        out_type=out_t, mesh=mesh, compiler_params=cp,
        scratch_types=[
            pltpu.VMEM((_W_CHUNK,), jnp.int32),
            pltpu.VMEM((1, _T), jnp.float32),
            pltpu.VMEM((1, _T), jnp.float32),
            pltpu.VMEM((2, _W_CHUNK), jnp.float32),
        ],
    )
    def sc_kernel(idx_hbm, sct_hbm, sht_hbm, params_hbm,
                  idx_v, sct_v, sht_v, params_v):
        wid = lax.axis_index("s") * 2 + lax.axis_index("c")
        base = wid * _W_CHUNK
        pltpu.sync_copy(idx_hbm.at[0, pl.ds(base, _W_CHUNK)], idx_v)
        pltpu.sync_copy(sct_hbm, sct_v)
        pltpu.sync_copy(sht_hbm, sht_v)
        z16 = jnp.zeros((16,), jnp.int32)

        @pl.loop(0, _W_CHUNK, step=16)
        def _(i):
            i16 = idx_v[pl.ds(i, 16)]
            params_v[0, pl.ds(i, 16)] = plsc.load_gather(sct_v, [z16, i16])
            params_v[1, pl.ds(i, 16)] = plsc.load_gather(sht_v, [z16, i16])

        pltpu.sync_copy(params_v, params_hbm.at[:, pl.ds(base, _W_CHUNK)])

    return sc_kernel(idx_pad, sct, sht)


def _tc_apply(pr_ref, x_ref, o_ref):
    d = x_ref.shape[1]
    ones = jnp.ones((1, d), jnp.float32)
    dn = (((0,), (0,)), ((), ()))
    scale = jax.lax.dot_general(pr_ref[0:1, :], ones, dn,
                                preferred_element_type=jnp.float32)
    shift = jax.lax.dot_general(pr_ref[1:2, :], ones, dn,
                                preferred_element_type=jnp.float32)
    o_ref[...] = scale * x_ref[...] + shift


def kernel(in_field, species_idx, scales, shifts):
    n, d = in_field.shape
    idx_pad = jnp.pad(species_idx.astype(jnp.int32), (0, _N_PAD - n)).reshape(1, _N_PAD)
    sct = scales.reshape(1, _T)
    sht = shifts.reshape(1, _T)
    params = _sc_gather(idx_pad, sct, sht)
    num_blocks = (n + _TC_B - 1) // _TC_B
    return pl.pallas_call(
        _tc_apply,
        grid=(num_blocks,),
        in_specs=[
            pl.BlockSpec((2, _TC_B), lambda i: (0, i)),
            pl.BlockSpec((_TC_B, d), lambda i: (i, 0)),
        ],
        out_specs=pl.BlockSpec((_TC_B, d), lambda i: (i, 0)),
        out_shape=jax.ShapeDtypeStruct((n, d), in_field.dtype),
        compiler_params=pltpu.CompilerParams(
            dimension_semantics=("parallel",),
        ),
    )(params, in_field)


# speedup vs baseline: 1.1142x; 1.0113x over previous
import dataclasses

import jax
import jax.numpy as jnp
from jax import lax
from jax.experimental import pallas as pl
from jax.experimental.pallas import tpu as pltpu
from jax.experimental.pallas import tpu_sc as plsc

_T = 64
_N_PAD = 102400
_W_CHUNK = 3200
_TC_B = 20480


def _sc_gather(idx_pad, sct, sht):
    mesh = plsc.VectorSubcoreMesh(core_axis_name="c", subcore_axis_name="s")
    out_t = jax.ShapeDtypeStruct((1, _N_PAD), jnp.float32)
    cp = pltpu.CompilerParams()
    if "needs_layout_passes" in pltpu.CompilerParams.__dataclass_fields__:
        cp = dataclasses.replace(cp, needs_layout_passes=False)

    @pl.kernel(
        out_type=(out_t, out_t), mesh=mesh, compiler_params=cp,
        scratch_types=[
            pltpu.VMEM((_W_CHUNK,), jnp.int32),
            pltpu.VMEM((1, _T), jnp.float32),
            pltpu.VMEM((1, _T), jnp.float32),
            pltpu.VMEM((_W_CHUNK,), jnp.float32),
            pltpu.VMEM((_W_CHUNK,), jnp.float32),
        ],
    )
    def sc_kernel(idx_hbm, sct_hbm, sht_hbm, scale_hbm, shift_hbm,
                  idx_v, sct_v, sht_v, scale_v, shift_v):
        wid = lax.axis_index("s") * 2 + lax.axis_index("c")
        base = wid * _W_CHUNK
        pltpu.sync_copy(idx_hbm.at[0, pl.ds(base, _W_CHUNK)], idx_v)
        pltpu.sync_copy(sct_hbm, sct_v)
        pltpu.sync_copy(sht_hbm, sht_v)
        z16 = jnp.zeros((16,), jnp.int32)

        @pl.loop(0, _W_CHUNK, step=16)
        def _(i):
            i16 = idx_v[pl.ds(i, 16)]
            scale_v[pl.ds(i, 16)] = plsc.load_gather(sct_v, [z16, i16])
            shift_v[pl.ds(i, 16)] = plsc.load_gather(sht_v, [z16, i16])

        pltpu.sync_copy(scale_v, scale_hbm.at[0, pl.ds(base, _W_CHUNK)])
        pltpu.sync_copy(shift_v, shift_hbm.at[0, pl.ds(base, _W_CHUNK)])

    return sc_kernel(idx_pad, sct, sht)


def _tc_apply(sc_ref, sh_ref, x_ref, o_ref):
    d = x_ref.shape[1]
    ones = jnp.ones((1, d), jnp.float32)
    dn = (((0,), (0,)), ((), ()))
    scale = jax.lax.dot_general(sc_ref[...], ones, dn,
                                preferred_element_type=jnp.float32)
    shift = jax.lax.dot_general(sh_ref[...], ones, dn,
                                preferred_element_type=jnp.float32)
    o_ref[...] = scale * x_ref[...] + shift


def kernel(in_field, species_idx, scales, shifts):
    n, d = in_field.shape
    idx_pad = jnp.pad(species_idx.astype(jnp.int32), (0, _N_PAD - n)).reshape(1, _N_PAD)
    sct = scales.reshape(1, _T)
    sht = shifts.reshape(1, _T)
    scale_row, shift_row = _sc_gather(idx_pad, sct, sht)
    num_blocks = (n + _TC_B - 1) // _TC_B
    return pl.pallas_call(
        _tc_apply,
        grid=(num_blocks,),
        in_specs=[
            pl.BlockSpec((1, _TC_B), lambda i: (0, i)),
            pl.BlockSpec((1, _TC_B), lambda i: (0, i)),
            pl.BlockSpec((_TC_B, d), lambda i: (i, 0)),
        ],
        out_specs=pl.BlockSpec((_TC_B, d), lambda i: (i, 0)),
        out_shape=jax.ShapeDtypeStruct((n, d), in_field.dtype),
        compiler_params=pltpu.CompilerParams(
            dimension_semantics=("parallel",),
        ),
    )(scale_row, shift_row, in_field)
